# baseline (speedup 1.0000x reference)
import functools

import jax
import jax.numpy as jnp
from jax import lax
from jax.experimental import pallas as pl
from jax.experimental.pallas import tpu as pltpu
from jax.experimental.pallas import tpu_sc as plsc

BATCH = 4
SEQ = 2048
D = 1024
NC = 2
NS = 16
NW = NC * NS
L = 16

POS_PER_W = SEQ // NW
CHUNK = 16
NBUF = 4
PF = 3
WPE_HALF = 32
N_HALF = POS_PER_W // WPE_HALF
CPH = WPE_HALF // CHUNK
NCHUNK = N_HALF * BATCH * CPH
NVEC = D // L

_mesh = plsc.VectorSubcoreMesh(core_axis_name="c", subcore_axis_name="s")


@functools.partial(
    pl.kernel,
    mesh=_mesh,
    out_type=jax.ShapeDtypeStruct((BATCH, SEQ, D), jnp.float32),
    scratch_types=[
        pltpu.VMEM((BATCH, POS_PER_W), jnp.int32),
        pltpu.VMEM((CHUNK, D), jnp.float32),
        pltpu.VMEM((CHUNK, D), jnp.float32),
        pltpu.VMEM((CHUNK, D), jnp.float32),
        pltpu.VMEM((CHUNK, D), jnp.float32),
        pltpu.VMEM((WPE_HALF, D), jnp.float32),
        pltpu.SemaphoreType.DMA,
        pltpu.SemaphoreType.DMA,
    ],
)
def _embed(ids_hbm, wte_hbm, wpe_hbm, out_hbm, ids_v, rows_a, rows_b, rows_c,
           rows_d, wpe_v, sem_g, sem_s):
    wid = lax.axis_index("s") * NC + lax.axis_index("c")
    p0 = wid * POS_PER_W

    for b in range(BATCH):
        pltpu.sync_copy(ids_hbm.at[b, pl.ds(p0, POS_PER_W)], ids_v.at[b])

    rows = [rows_a, rows_b, rows_c, rows_d]

    def chunk_coords(t):
        h, rem = divmod(t, BATCH * CPH)
        b, c = divmod(rem, CPH)
        return h, b, c

    def start_gather(t):
        h, b, c = chunk_coords(t)
        off = h * WPE_HALF + c * CHUNK
        return pltpu.async_copy(
            wte_hbm.at[ids_v.at[b, pl.ds(off, CHUNK)]], rows[t % NBUF], sem_g)

    def start_store(t):
        h, b, c = chunk_coords(t)
        off = h * WPE_HALF + c * CHUNK
        return pltpu.async_copy(
            rows[t % NBUF], out_hbm.at[b, pl.ds(p0 + off, CHUNK)], sem_s)

    gathers = [None] * NCHUNK
    stores = [None] * NCHUNK

    for t in range(min(PF, NCHUNK)):
        gathers[t] = start_gather(t)

    for t in range(NCHUNK):
        tp = t + PF
        if tp < NCHUNK:
            if tp - NBUF >= 0:
                stores[tp - NBUF].wait()
            gathers[tp] = start_gather(tp)
        gathers[t].wait()
        h, b, c = chunk_coords(t)
        if b == 0 and c == 0:
            pltpu.sync_copy(wpe_hbm.at[pl.ds(p0 + h * WPE_HALF, WPE_HALF)],
                            wpe_v)
        buf = rows[t % NBUF]
        wofs = c * CHUNK

        def add_row(i, carry):
            for j in range(NVEC):
                plsc.addupdate(buf.at[i, pl.ds(j * L, L)],
                               wpe_v[wofs + i, pl.ds(j * L, L)])
            return carry

        lax.fori_loop(0, CHUNK, add_row, 0)
        stores[t] = start_store(t)

    for t in range(NCHUNK - NBUF, NCHUNK):
        stores[t].wait()


def kernel(input_ids, wte, wpe):
    return _embed(input_ids.astype(jnp.int32), wte, wpe)


# speedup vs baseline: 1.1338x; 1.1338x over previous
import functools

import jax
import jax.numpy as jnp
from jax import lax
from jax.experimental import pallas as pl
from jax.experimental.pallas import tpu as pltpu
from jax.experimental.pallas import tpu_sc as plsc

BATCH = 4
SEQ = 2048
D = 1024
NC = 2
NS = 16
NW = NC * NS
L = 16

POS_PER_W = SEQ // NW
CHUNK = 32
N_HALF = POS_PER_W // CHUNK
NCHUNK = N_HALF * BATCH
VECS = CHUNK * (D // L)

_mesh = plsc.VectorSubcoreMesh(core_axis_name="c", subcore_axis_name="s")


@functools.partial(
    pl.kernel,
    mesh=_mesh,
    out_type=jax.ShapeDtypeStruct((BATCH, SEQ, D), jnp.float32),
    scratch_types=[
        pltpu.VMEM((BATCH, POS_PER_W), jnp.int32),
        pltpu.VMEM((CHUNK, D), jnp.float32),
        pltpu.VMEM((CHUNK, D), jnp.float32),
        pltpu.VMEM((CHUNK, D), jnp.float32),
        pltpu.SemaphoreType.DMA,
        pltpu.SemaphoreType.DMA,
    ],
)
def _embed(ids_hbm, wte_hbm, wpe_hbm, out_hbm, ids_v, rows_a, rows_b, wpe_v,
           sem_g, sem_s):
    wid = lax.axis_index("s") * NC + lax.axis_index("c")
    p0 = wid * POS_PER_W

    for b in range(BATCH):
        pltpu.sync_copy(ids_hbm.at[b, pl.ds(p0, POS_PER_W)], ids_v.at[b])

    rows = [rows_a, rows_b]

    def chunk_coords(t):
        h, b = divmod(t, BATCH)
        return h, b

    def start_gather(t):
        h, b = chunk_coords(t)
        return pltpu.async_copy(
            wte_hbm.at[ids_v.at[b, pl.ds(h * CHUNK, CHUNK)]],
            rows[t % 2], sem_g)

    def start_store(t):
        h, b = chunk_coords(t)
        return pltpu.async_copy(
            rows[t % 2], out_hbm.at[b, pl.ds(p0 + h * CHUNK, CHUNK)], sem_s)

    gathers = [None] * NCHUNK
    stores = [None] * NCHUNK

    gathers[0] = start_gather(0)
    for t in range(NCHUNK):
        if t + 1 < NCHUNK:
            if t - 1 >= 0:
                stores[t - 1].wait()
            gathers[t + 1] = start_gather(t + 1)
        gathers[t].wait()
        h, b = chunk_coords(t)
        if b == 0:
            pltpu.sync_copy(wpe_hbm.at[pl.ds(p0 + h * CHUNK, CHUNK)], wpe_v)
        buf = rows[t % 2]

        def add_row(i, carry):
            for j in range(D // L):
                plsc.addupdate(buf.at[i, pl.ds(j * L, L)],
                               wpe_v[i, pl.ds(j * L, L)])
            return carry

        lax.fori_loop(0, CHUNK, add_row, 0)
        stores[t] = start_store(t)
    stores[NCHUNK - 2].wait()
    stores[NCHUNK - 1].wait()


def kernel(input_ids, wte, wpe):
    return _embed(input_ids.astype(jnp.int32), wte, wpe)


# speedup vs baseline: 1.2696x; 1.1198x over previous
import functools

import jax
import jax.numpy as jnp
from jax import lax
from jax.experimental import pallas as pl
from jax.experimental.pallas import tpu as pltpu
from jax.experimental.pallas import tpu_sc as plsc

BATCH = 4
SEQ = 2048
D = 1024
NC = 2
NS = 16
NW = NC * NS
L = 16

POS_PER_W = SEQ // NW
CHUNK = 32
N_HALF = POS_PER_W // CHUNK
NCHUNK = N_HALF * BATCH
VECS = CHUNK * (D // L)

_mesh = plsc.VectorSubcoreMesh(core_axis_name="c", subcore_axis_name="s")


@functools.partial(
    pl.kernel,
    mesh=_mesh,
    out_type=jax.ShapeDtypeStruct((BATCH, SEQ, D), jnp.float32),
    scratch_types=[
        pltpu.VMEM((BATCH, POS_PER_W), jnp.int32),
        pltpu.VMEM((CHUNK, D), jnp.float32),
        pltpu.VMEM((CHUNK, D), jnp.float32),
        pltpu.VMEM((CHUNK, D), jnp.float32),
        pltpu.SemaphoreType.DMA,
        pltpu.SemaphoreType.DMA,
    ],
)
def _embed(ids_hbm, wte_hbm, wpe_hbm, out_hbm, ids_v, rows_a, rows_b, wpe_v,
           sem_g, sem_s):
    wid = lax.axis_index("s") * NC + lax.axis_index("c")
    p0 = wid * POS_PER_W

    for b in range(BATCH):
        pltpu.sync_copy(ids_hbm.at[b, pl.ds(p0, POS_PER_W)], ids_v.at[b])

    rows = [rows_a, rows_b]

    def chunk_coords(t):
        h, b = divmod(t, BATCH)
        return h, b

    def start_gather(t):
        h, b = chunk_coords(t)
        return pltpu.async_copy(
            wte_hbm.at[ids_v.at[b, pl.ds(h * CHUNK, CHUNK)]],
            rows[t % 2], sem_g)

    def start_store(t):
        h, b = chunk_coords(t)
        return pltpu.async_copy(
            rows[t % 2], out_hbm.at[b, pl.ds(p0 + h * CHUNK, CHUNK)], sem_s)

    gathers = [None] * NCHUNK
    stores = [None] * NCHUNK

    gathers[0] = start_gather(0)
    for t in range(NCHUNK):
        if t + 1 < NCHUNK:
            if t - 1 >= 0:
                stores[t - 1].wait()
            gathers[t + 1] = start_gather(t + 1)
        gathers[t].wait()
        h, b = chunk_coords(t)
        if b == 0:
            pltpu.sync_copy(wpe_hbm.at[pl.ds(p0 + h * CHUNK, CHUNK)], wpe_v)
        buf = rows[t % 2]

        def add_body(k, carry):
            i = k >> 6
            j = pl.multiple_of((k & 63) << 4, L)
            plsc.addupdate(buf.at[i, pl.ds(j, L)], wpe_v[i, pl.ds(j, L)])
            return carry

        lax.fori_loop(0, VECS, add_body, 0, unroll=8)
        stores[t] = start_store(t)
    stores[NCHUNK - 2].wait()
    stores[NCHUNK - 1].wait()


def kernel(input_ids, wte, wpe):
    return _embed(input_ids.astype(jnp.int32), wte, wpe)


# speedup vs baseline: 2.0925x; 1.6481x over previous
import functools

import jax
import jax.numpy as jnp
from jax import lax
from jax.experimental import pallas as pl
from jax.experimental.pallas import tpu as pltpu
from jax.experimental.pallas import tpu_sc as plsc

BATCH = 4
SEQ = 2048
D = 1024
NC = 2
NS = 16
NW = NC * NS
L = 16

POS_PER_W = SEQ // NW
CHUNK = 32
N_HALF = POS_PER_W // CHUNK
NCHUNK = N_HALF * BATCH
VECS = CHUNK * (D // L)

_mesh = plsc.VectorSubcoreMesh(core_axis_name="c", subcore_axis_name="s")


@functools.partial(
    pl.kernel,
    mesh=_mesh,
    out_type=jax.ShapeDtypeStruct((BATCH, SEQ, D), jnp.float32),
    scratch_types=[
        pltpu.VMEM((BATCH, POS_PER_W), jnp.int32),
        pltpu.VMEM((CHUNK, D), jnp.float32),
        pltpu.VMEM((CHUNK, D), jnp.float32),
        pltpu.VMEM((CHUNK, D), jnp.float32),
        pltpu.VMEM_SHARED((NS, CHUNK // 2, D), jnp.float32),
        pltpu.SemaphoreType.DMA,
        pltpu.SemaphoreType.DMA,
        pltpu.SemaphoreType.DMA,
        pltpu.SemaphoreType.DMA,
    ],
)
def _embed(ids_hbm, wte_hbm, wpe_hbm, out_hbm, ids_v, rows_a, rows_b, wpe_v,
           stage_sh, sem_g, sem_s, sem_x, sem_p):
    cid = lax.axis_index("c")
    sid = lax.axis_index("s")
    wid = sid * NC + cid
    p0 = wid * POS_PER_W

    for b in range(BATCH):
        pltpu.sync_copy(ids_hbm.at[b, pl.ds(p0, POS_PER_W)], ids_v.at[b])

    rows = [rows_a, rows_b]

    def chunk_coords(t):
        h, b = divmod(t, BATCH)
        return h, b

    def start_gather(t):
        h, b = chunk_coords(t)
        return pltpu.async_copy(
            wte_hbm.at[ids_v.at[b, pl.ds(h * CHUNK, CHUNK)]],
            rows[t % 2], sem_g)

    HALF = CHUNK // 2

    def start_store_hi(t):
        h, b = chunk_coords(t)
        return pltpu.async_copy(
            rows[t % 2].at[pl.ds(HALF, HALF)],
            out_hbm.at[b, pl.ds(p0 + h * CHUNK + HALF, HALF)], sem_s)

    def start_store_lo(t):
        h, b = chunk_coords(t)
        return pltpu.async_copy(
            stage_sh.at[sid],
            out_hbm.at[b, pl.ds(p0 + h * CHUNK, HALF)], sem_p)

    gathers = [None] * NCHUNK
    stores_hi = [None] * NCHUNK
    stores_lo = [None] * NCHUNK
    stages = [None] * NCHUNK

    gathers[0] = start_gather(0)
    for t in range(NCHUNK):
        if t - 1 >= 0:
            stages[t - 1].wait()
            stores_lo[t - 1] = start_store_lo(t - 1)
        if t + 1 < NCHUNK:
            if t - 1 >= 0:
                stores_hi[t - 1].wait()
            gathers[t + 1] = start_gather(t + 1)
        gathers[t].wait()
        h, b = chunk_coords(t)
        if b == 0:
            pltpu.sync_copy(wpe_hbm.at[pl.ds(p0 + h * CHUNK, CHUNK)], wpe_v)
        buf = rows[t % 2]

        def add_body(k, carry):
            i = k >> 6
            j = pl.multiple_of((k & 63) << 4, L)
            plsc.addupdate(buf.at[i, pl.ds(j, L)], wpe_v[i, pl.ds(j, L)])
            return carry

        lax.fori_loop(0, VECS, add_body, 0, unroll=8)
        stores_hi[t] = start_store_hi(t)
        if t - 1 >= 0:
            stores_lo[t - 1].wait()
        stages[t] = pltpu.async_copy(buf.at[pl.ds(0, HALF)], stage_sh.at[sid],
                                     sem_x)
    stages[NCHUNK - 1].wait()
    stores_lo[NCHUNK - 1] = start_store_lo(NCHUNK - 1)
    stores_hi[NCHUNK - 1].wait()
    stores_lo[NCHUNK - 1].wait()


def kernel(input_ids, wte, wpe):
    return _embed(input_ids.astype(jnp.int32), wte, wpe)
